# initial kernel scaffold (unmeasured)
import jax
import jax.numpy as jnp
from jax import lax
from jax.experimental import pallas as pl
from jax.experimental.pallas import tpu as pltpu

N_DEV = 32
M_PER = 128
N_OUT = 2048


def _e4m3_quant_dequant(y, scale):
    z = y / scale
    a = jnp.abs(z)
    sgn = jnp.sign(z)
    bits = lax.bitcast_convert_type(a, jnp.uint32)
    p2 = lax.bitcast_convert_type(
        bits & jnp.uint32(0xFF800000), jnp.float32
    )
    step = jnp.where(a >= 2.0 ** -6, p2 * 0.125, 2.0 ** -9)
    q = jnp.round(a / step) * step
    q = jnp.minimum(q, 448.0)
    return sgn * q * scale


def kernel(x, w_mat):
    def body(x_ref, w_ref, out_ref, comm_ref, gather_ref, stage_ref,
             ring_send, ring_recv, amax_send, amax_recv, bcast_recv):
        d = lax.axis_index("i")
        left = (d - 1) % N_DEV
        right = (d + 1) % N_DEV

        barrier_sem = pltpu.get_barrier_semaphore()
        for nbr in (left, right):
            pl.semaphore_signal(
                barrier_sem, inc=1,
                device_id=(nbr,), device_id_type=pl.DeviceIdType.MESH,
            )
        pl.semaphore_wait(barrier_sem, 2)

        def gemm(k):
            xk = x_ref[pl.ds(k * M_PER, M_PER), :]
            return jnp.dot(xk, w_ref[...], preferred_element_type=jnp.float32)

        acc = gemm((d - 1) % N_DEV)
        for s in range(N_DEV - 1):
            send_slot = s % 2
            recv_slot = (s + 1) % 2
            comm_ref[send_slot] = acc.astype(jnp.bfloat16)
            rdma = pltpu.make_async_remote_copy(
                src_ref=comm_ref.at[send_slot],
                dst_ref=comm_ref.at[recv_slot],
                send_sem=ring_send.at[send_slot],
                recv_sem=ring_recv.at[recv_slot],
                device_id=(right,),
                device_id_type=pl.DeviceIdType.MESH,
            )
            rdma.start()
            rdma.wait()
            kr = (d - 2 - s) % N_DEV
            acc = comm_ref[recv_slot].astype(jnp.float32) + gemm(kr)

        local_amax = jnp.max(jnp.abs(acc))
        stage_ref[...] = jnp.full((8, 128), local_amax, jnp.float32)

        @pl.when(d != 0)
        def _():
            g = pltpu.make_async_remote_copy(
                src_ref=stage_ref,
                dst_ref=gather_ref.at[d],
                send_sem=amax_send,
                recv_sem=amax_recv,
                device_id=(0,),
                device_id_type=pl.DeviceIdType.MESH,
            )
            g.start()
            g.wait_send()
            b = pltpu.make_async_remote_copy(
                src_ref=stage_ref, dst_ref=stage_ref,
                send_sem=amax_send, recv_sem=bcast_recv,
                device_id=(0,), device_id_type=pl.DeviceIdType.MESH,
            )
            b.wait_recv()

        @pl.when(d == 0)
        def _():
            gather_ref[0] = stage_ref[...]
            rwait = pltpu.make_async_remote_copy(
                src_ref=stage_ref, dst_ref=gather_ref.at[0],
                send_sem=amax_send, recv_sem=amax_recv,
                device_id=(0,), device_id_type=pl.DeviceIdType.MESH,
            )
            for _ in range(N_DEV - 1):
                rwait.wait_recv()
            gmax = jnp.max(gather_ref[...])
            stage_ref[...] = jnp.full((8, 128), gmax, jnp.float32)
            snd = None
            for t in range(1, N_DEV):
                snd = pltpu.make_async_remote_copy(
                    src_ref=stage_ref, dst_ref=stage_ref,
                    send_sem=amax_send, recv_sem=bcast_recv,
                    device_id=(t,), device_id_type=pl.DeviceIdType.MESH,
                )
                snd.start()
            for _ in range(N_DEV - 1):
                snd.wait_send()

        gmax = stage_ref[0, 0]
        scale = gmax / 448.0
        out_ref[...] = _e4m3_quant_dequant(acc, scale)

    return pl.pallas_call(
        body,
        out_shape=jax.ShapeDtypeStruct((M_PER, N_OUT), jnp.float32),
        in_specs=[
            pl.BlockSpec(memory_space=pltpu.VMEM),
            pl.BlockSpec(memory_space=pltpu.VMEM),
        ],
        out_specs=pl.BlockSpec(memory_space=pltpu.VMEM),
        scratch_shapes=[
            pltpu.VMEM((2, M_PER, N_OUT), jnp.bfloat16),
            pltpu.VMEM((N_DEV, 8, 128), jnp.float32),
            pltpu.VMEM((8, 128), jnp.float32),
            pltpu.SemaphoreType.DMA((2,)),
            pltpu.SemaphoreType.DMA((2,)),
            pltpu.SemaphoreType.DMA,
            pltpu.SemaphoreType.DMA,
            pltpu.SemaphoreType.DMA,
        ],
        compiler_params=pltpu.CompilerParams(collective_id=0),
    )(x, w_mat)


# baseline (device time: 269508 ns/iter reference)
import jax
import jax.numpy as jnp
from jax import lax
from jax.experimental import pallas as pl
from jax.experimental.pallas import tpu as pltpu

N_DEV = 32
M_PER = 128
K_PER = 128
N_OUT = 2048


def _e4m3_quant_dequant(y, scale):
    z = y / scale
    a = jnp.abs(z)
    sgn = jnp.sign(z)
    bits = lax.bitcast_convert_type(a, jnp.uint32)
    p2 = lax.bitcast_convert_type(
        bits & jnp.uint32(0xFF800000), jnp.float32
    )
    step = jnp.where(a >= 2.0 ** -6, p2 * 0.125, 2.0 ** -9)
    q = jnp.round(a / step) * step
    q = jnp.minimum(q, 448.0)
    return sgn * q * scale


def kernel(x, w_mat):
    def body(x_ref, w_ref, out_ref, xg_ref, wcomm_ref, xbf_ref,
             gather_ref, stage_ref,
             ring_send, ring_recv, x_send, x_recv,
             amax_send, amax_recv, bcast_recv):
        d = lax.axis_index("i")
        left = (d - 1) % N_DEV
        right = (d + 1) % N_DEV

        barrier_sem = pltpu.get_barrier_semaphore()
        for t in range(N_DEV):
            @pl.when(d != t)
            def _():
                pl.semaphore_signal(
                    barrier_sem, inc=1,
                    device_id=(t,), device_id_type=pl.DeviceIdType.MESH,
                )
        pl.semaphore_wait(barrier_sem, N_DEV - 1)

        xbf_ref[...] = x_ref[...].astype(jnp.bfloat16)
        xg_ref[d] = xbf_ref[pl.ds(d * M_PER, M_PER), :]
        for off in range(1, N_DEV):
            t = (d + off) % N_DEV
            xs = pltpu.make_async_remote_copy(
                src_ref=xbf_ref.at[pl.ds(t * M_PER, M_PER), :],
                dst_ref=xg_ref.at[d],
                send_sem=x_send,
                recv_sem=x_recv,
                device_id=(t,),
                device_id_type=pl.DeviceIdType.MESH,
            )
            xs.start()
        xr = pltpu.make_async_remote_copy(
            src_ref=xbf_ref.at[pl.ds(0, M_PER), :],
            dst_ref=xg_ref.at[0],
            send_sem=x_send,
            recv_sem=x_recv,
            device_id=(0,),
            device_id_type=pl.DeviceIdType.MESH,
        )
        for _ in range(N_DEV - 1):
            xr.wait_recv()

        wcomm_ref[0] = w_ref[...].astype(jnp.bfloat16)
        acc = jnp.dot(
            xg_ref[d], wcomm_ref[0], preferred_element_type=jnp.float32
        )
        for s in range(N_DEV - 1):
            send_slot = s % 2
            recv_slot = (s + 1) % 2
            rdma = pltpu.make_async_remote_copy(
                src_ref=wcomm_ref.at[send_slot],
                dst_ref=wcomm_ref.at[recv_slot],
                send_sem=ring_send.at[send_slot],
                recv_sem=ring_recv.at[recv_slot],
                device_id=(right,),
                device_id_type=pl.DeviceIdType.MESH,
            )
            rdma.start()
            rdma.wait()
            c = (d - 1 - s) % N_DEV
            acc = acc + jnp.dot(
                xg_ref[c], wcomm_ref[recv_slot],
                preferred_element_type=jnp.float32,
            )

        for _ in range(N_DEV - 1):
            xr.wait_send()

        local_amax = jnp.max(jnp.abs(acc))
        stage_ref[...] = jnp.full((8, 128), local_amax, jnp.float32)

        @pl.when(d != 0)
        def _():
            g = pltpu.make_async_remote_copy(
                src_ref=stage_ref,
                dst_ref=gather_ref.at[d],
                send_sem=amax_send,
                recv_sem=amax_recv,
                device_id=(0,),
                device_id_type=pl.DeviceIdType.MESH,
            )
            g.start()
            g.wait_send()
            b = pltpu.make_async_remote_copy(
                src_ref=stage_ref, dst_ref=stage_ref,
                send_sem=amax_send, recv_sem=bcast_recv,
                device_id=(0,), device_id_type=pl.DeviceIdType.MESH,
            )
            b.wait_recv()

        @pl.when(d == 0)
        def _():
            gather_ref[0] = stage_ref[...]
            rwait = pltpu.make_async_remote_copy(
                src_ref=stage_ref, dst_ref=gather_ref.at[0],
                send_sem=amax_send, recv_sem=amax_recv,
                device_id=(0,), device_id_type=pl.DeviceIdType.MESH,
            )
            for _ in range(N_DEV - 1):
                rwait.wait_recv()
            gmax = jnp.max(gather_ref[...])
            stage_ref[...] = jnp.full((8, 128), gmax, jnp.float32)
            snd = None
            for t in range(1, N_DEV):
                snd = pltpu.make_async_remote_copy(
                    src_ref=stage_ref, dst_ref=stage_ref,
                    send_sem=amax_send, recv_sem=bcast_recv,
                    device_id=(t,), device_id_type=pl.DeviceIdType.MESH,
                )
                snd.start()
            for _ in range(N_DEV - 1):
                snd.wait_send()

        gmax = stage_ref[0, 0]
        scale = gmax / 448.0
        out_ref[...] = _e4m3_quant_dequant(acc, scale)

    return pl.pallas_call(
        body,
        out_shape=jax.ShapeDtypeStruct((M_PER, N_OUT), jnp.float32),
        in_specs=[
            pl.BlockSpec(memory_space=pltpu.VMEM),
            pl.BlockSpec(memory_space=pltpu.VMEM),
        ],
        out_specs=pl.BlockSpec(memory_space=pltpu.VMEM),
        scratch_shapes=[
            pltpu.VMEM((N_DEV, M_PER, K_PER), jnp.bfloat16),
            pltpu.VMEM((2, K_PER, N_OUT), jnp.bfloat16),
            pltpu.VMEM((N_DEV * M_PER, K_PER), jnp.bfloat16),
            pltpu.VMEM((N_DEV, 8, 128), jnp.float32),
            pltpu.VMEM((8, 128), jnp.float32),
            pltpu.SemaphoreType.DMA((2,)),
            pltpu.SemaphoreType.DMA((2,)),
            pltpu.SemaphoreType.DMA,
            pltpu.SemaphoreType.DMA,
            pltpu.SemaphoreType.DMA,
            pltpu.SemaphoreType.DMA,
            pltpu.SemaphoreType.DMA,
        ],
        compiler_params=pltpu.CompilerParams(collective_id=0),
    )(x, w_mat)
